# Initial kernel scaffold; baseline (speedup 1.0000x reference)
import jax
import jax.numpy as jnp
from jax.experimental import pallas as pl


def kernel(x, edge_index, W1, b1, g1, be1, W2, b2, g2, be2, W3, b3, g3, be3, W4, b4):
    raise NotImplementedError("write your pallas kernel here")


# speedup vs baseline: 16.3100x; 16.3100x over previous
import functools

import jax
import jax.numpy as jnp
from jax import lax
from jax.experimental import pallas as pl
from jax.experimental.pallas import tpu as pltpu
from jax.experimental.pallas import tpu_sc as plsc

N_NODES = 10000
N_PAD = 10240
E = 320000
NCORE = 2
NTILE = 16
E_PT = E // (NCORE * NTILE)
CHUNK = 1000
ZROWS = 64

_f32 = jnp.float32


@functools.lru_cache(maxsize=None)
def _sc_scatter_gather(D):
    mesh = plsc.VectorSubcoreMesh(core_axis_name="c", subcore_axis_name="s")
    rows_pt = N_PAD // NTILE
    nchunk = E_PT // CHUNK
    nz = rows_pt // ZROWS

    def body(u_hbm, g_hbm, s_hbm, z_hbm, out_hbm,
             idx_g, idx_s, rows, zbuf, acc, sem):
        cid = lax.axis_index("c")
        sid = lax.axis_index("s")
        base = sid * rows_pt
        pltpu.sync_copy(z_hbm, zbuf)
        for j in range(nz):
            pltpu.sync_copy(zbuf, acc.at[pl.ds(base + j * ZROWS, ZROWS)])
        plsc.subcore_barrier()
        ebase = (cid * NTILE + sid) * E_PT
        for i in range(nchunk):
            eb = ebase + i * CHUNK
            pltpu.sync_copy(g_hbm.at[pl.ds(eb, CHUNK)], idx_g)
            pltpu.sync_copy(s_hbm.at[pl.ds(eb, CHUNK)], idx_s)
            pltpu.async_copy(u_hbm.at[idx_g], rows, sem).wait()
            pltpu.sync_copy(rows, acc.at[idx_s], add=True)
        plsc.subcore_barrier()
        pltpu.sync_copy(acc.at[pl.ds(base, rows_pt)],
                        out_hbm.at[cid, pl.ds(base, rows_pt)])

    return pl.kernel(
        body,
        out_type=jax.ShapeDtypeStruct((NCORE, N_PAD, D), _f32),
        mesh=mesh,
        scratch_types=[
            pltpu.VMEM((CHUNK,), jnp.int32),
            pltpu.VMEM((CHUNK,), jnp.int32),
            pltpu.VMEM((CHUNK, D), _f32),
            pltpu.VMEM((ZROWS, D), _f32),
            pltpu.VMEM_SHARED((N_PAD, D), _f32),
            pltpu.SemaphoreType.DMA,
        ],
        compiler_params=pltpu.CompilerParams(use_tc_tiling_on_sc=False),
    )


_BN = 2048


def _mm_all(h, W):
    K, din, dout = W.shape
    nb = N_PAD // _BN

    def body(h_ref, w_ref, y_ref):
        y_ref[0] = jnp.dot(h_ref[...], w_ref[0],
                           preferred_element_type=_f32)

    return pl.pallas_call(
        body,
        grid=(K, nb),
        in_specs=[pl.BlockSpec((_BN, din), lambda k, n: (n, 0)),
                  pl.BlockSpec((1, din, dout), lambda k, n: (k, 0, 0))],
        out_specs=pl.BlockSpec((1, _BN, dout), lambda k, n: (k, n, 0)),
        out_shape=jax.ShapeDtypeStruct((K, N_PAD, dout), _f32),
    )(h, W)


def _scale(dinv, y):
    D = y.shape[1]
    nb = N_PAD // _BN

    def body(d_ref, y_ref, u_ref):
        u_ref[...] = d_ref[...] * y_ref[...]

    return pl.pallas_call(
        body,
        grid=(nb,),
        in_specs=[pl.BlockSpec((_BN, 1), lambda n: (n, 0)),
                  pl.BlockSpec((_BN, D), lambda n: (n, 0))],
        out_specs=pl.BlockSpec((_BN, D), lambda n: (n, 0)),
        out_shape=jax.ShapeDtypeStruct((N_PAD, D), _f32),
    )(dinv, y)


def _step(dinv, y, p, b2):
    D = y.shape[1]
    nb = N_PAD // _BN

    def body(d_ref, y_ref, p_ref, b2_ref, b_ref, u_ref):
        s = p_ref[0] + p_ref[1]
        b = y_ref[...] - 2.0 * d_ref[...] * s - b2_ref[...]
        b_ref[...] = b
        u_ref[...] = d_ref[...] * b

    return pl.pallas_call(
        body,
        grid=(nb,),
        in_specs=[pl.BlockSpec((_BN, 1), lambda n: (n, 0)),
                  pl.BlockSpec((_BN, D), lambda n: (n, 0)),
                  pl.BlockSpec((2, _BN, D), lambda n: (0, n, 0)),
                  pl.BlockSpec((_BN, D), lambda n: (n, 0))],
        out_specs=[pl.BlockSpec((_BN, D), lambda n: (n, 0)),
                   pl.BlockSpec((_BN, D), lambda n: (n, 0))],
        out_shape=[jax.ShapeDtypeStruct((N_PAD, D), _f32),
                   jax.ShapeDtypeStruct((N_PAD, D), _f32)],
    )(dinv, y, p, b2)


def _final(dinv, y, p, b2, gg, be, relu):
    D = y.shape[1]
    nb = N_PAD // _BN

    def body(d_ref, y_ref, p_ref, b2_ref, gg_ref, be_ref, o_ref):
        s = p_ref[0] + p_ref[1]
        o = gg_ref[...] * (y_ref[...] - d_ref[...] * s - b2_ref[...]) \
            + be_ref[...]
        o_ref[...] = jnp.maximum(o, 0.0) if relu else o

    return pl.pallas_call(
        body,
        grid=(nb,),
        in_specs=[pl.BlockSpec((_BN, 1), lambda n: (n, 0)),
                  pl.BlockSpec((_BN, D), lambda n: (n, 0)),
                  pl.BlockSpec((2, _BN, D), lambda n: (0, n, 0)),
                  pl.BlockSpec((_BN, D), lambda n: (n, 0)),
                  pl.BlockSpec((1, D), lambda n: (0, 0)),
                  pl.BlockSpec((1, D), lambda n: (0, 0))],
        out_specs=pl.BlockSpec((_BN, D), lambda n: (n, 0)),
        out_shape=jax.ShapeDtypeStruct((N_PAD, D), _f32),
    )(dinv, y, p, b2, gg, be)


def _pad_to(a, shape):
    pads = [(0, t - s) for s, t in zip(a.shape, shape)]
    return jnp.pad(a, pads)


def _cheb_layer(h, W, bias, g, be, dinv, src, dst, zbuf_hbm, dout_pad, relu):
    K = W.shape[0]
    Wp = _pad_to(W, (K, h.shape[1], dout_pad))
    y = _mm_all(h, Wp)
    if g is not None:
        gg = g / jnp.sqrt(1.0 + 1e-5)
        bev = gg * bias + be
    else:
        gg = jnp.ones_like(bias)
        bev = bias
    gg = _pad_to(gg, (dout_pad,)).reshape(1, dout_pad)
    bev = _pad_to(bev, (dout_pad,)).reshape(1, dout_pad)

    lap = _sc_scatter_gather(dout_pad)
    u = _scale(dinv, y[K - 1])
    bk2 = jnp.zeros((N_PAD, dout_pad), _f32)
    bk1 = y[K - 1]
    for k in range(K - 2, 0, -1):
        p = lap(u, src, dst, zbuf_hbm)
        b, u = _step(dinv, y[k], p, bk2)
        bk2, bk1 = bk1, b
    p = lap(u, src, dst, zbuf_hbm)
    return _final(dinv, y[0], p, bk2, gg, bev, relu)


def kernel(x, edge_index, W1, b1, g1, be1, W2, b2, g2, be2,
           W3, b3, g3, be3, W4, b4):
    src = edge_index[0]
    dst = edge_index[1]
    xp = _pad_to(x, (N_PAD, x.shape[1]))

    z16 = jnp.zeros((ZROWS, 16), _f32)
    ones = jnp.ones((N_PAD, 16), _f32)
    pdeg = _sc_scatter_gather(16)(ones, dst, src, z16)
    deg = pdeg[0, :, 0] + pdeg[1, :, 0]
    dinv = jnp.where(deg > 0, lax.rsqrt(jnp.maximum(deg, 1e-12)), 0.0)
    dinv = dinv[:, None]

    z32 = jnp.zeros((ZROWS, 32), _f32)
    z64 = jnp.zeros((ZROWS, 64), _f32)

    h = _cheb_layer(xp, W1, b1, g1, be1, dinv, src, dst, z64, 64, True)
    h = _cheb_layer(h, W2, b2, g2, be2, dinv, src, dst, z32, 32, True)
    h = _cheb_layer(h, W3, b3, g3, be3, dinv, src, dst, z16, 16, True)
    h = _cheb_layer(h, W4, b4, None, None, dinv, src, dst, z16, 16, False)
    return h[:N_NODES, :10]


# speedup vs baseline: 19.1998x; 1.1772x over previous
import functools

import jax
import jax.numpy as jnp
from jax import lax
from jax.experimental import pallas as pl
from jax.experimental.pallas import tpu as pltpu
from jax.experimental.pallas import tpu_sc as plsc

N_NODES = 10000
N_PAD = 10240
E = 320000
NCORE = 2
NTILE = 16
E_PT = E // (NCORE * NTILE)
ZROWS = 64

_f32 = jnp.float32


@functools.lru_cache(maxsize=None)
def _sc_scatter_gather(D):
    mesh = plsc.VectorSubcoreMesh(core_axis_name="c", subcore_axis_name="s")
    rows_pt = N_PAD // NTILE
    chunk = 400 if D >= 64 else 1000
    nchunk = E_PT // chunk
    nz = rows_pt // ZROWS

    def body(u_hbm, g_hbm, s_hbm, z_hbm, out_hbm,
             ig0, is0, rows0, ig1, is1, rows1, zbuf, acc,
             sg0, ss0, sg1, ss1):
        cid = lax.axis_index("c")
        sid = lax.axis_index("s")
        base = sid * rows_pt
        ebase = (cid * NTILE + sid) * E_PT
        bufs = [(ig0, is0, rows0, sg0, ss0), (ig1, is1, rows1, sg1, ss1)]
        gath = [None, None]
        pend = [None, None]

        def load(i):
            s = i % 2
            ig, is_, rows, sg, _ = bufs[s]
            if pend[s] is not None:
                pend[s].wait()
                pend[s] = None
            eb = ebase + i * chunk
            pltpu.sync_copy(g_hbm.at[pl.ds(eb, chunk)], ig)
            pltpu.sync_copy(s_hbm.at[pl.ds(eb, chunk)], is_)
            gath[s] = pltpu.async_copy(u_hbm.at[ig], rows, sg)

        load(0)
        pltpu.sync_copy(z_hbm, zbuf)
        for j in range(nz):
            pltpu.sync_copy(zbuf, acc.at[pl.ds(base + j * ZROWS, ZROWS)])
        plsc.subcore_barrier()
        for i in range(nchunk):
            s = i % 2
            if i + 1 < nchunk:
                load(i + 1)
            _, is_, rows, _, ss = bufs[s]
            gath[s].wait()
            pend[s] = pltpu.async_copy(rows, acc.at[is_], ss, add=True)
        for p in pend:
            if p is not None:
                p.wait()
        plsc.subcore_barrier()
        pltpu.sync_copy(acc.at[pl.ds(base, rows_pt)],
                        out_hbm.at[cid, pl.ds(base, rows_pt)])

    return pl.kernel(
        body,
        out_type=jax.ShapeDtypeStruct((NCORE, N_PAD, D), _f32),
        mesh=mesh,
        scratch_types=[
            pltpu.VMEM((chunk,), jnp.int32),
            pltpu.VMEM((chunk,), jnp.int32),
            pltpu.VMEM((chunk, D), _f32),
            pltpu.VMEM((chunk,), jnp.int32),
            pltpu.VMEM((chunk,), jnp.int32),
            pltpu.VMEM((chunk, D), _f32),
            pltpu.VMEM((ZROWS, D), _f32),
            pltpu.VMEM_SHARED((N_PAD, D), _f32),
            pltpu.SemaphoreType.DMA,
            pltpu.SemaphoreType.DMA,
            pltpu.SemaphoreType.DMA,
            pltpu.SemaphoreType.DMA,
        ],
        compiler_params=pltpu.CompilerParams(use_tc_tiling_on_sc=False),
    )


def _sc_degree():
    mesh = plsc.VectorSubcoreMesh(core_axis_name="c", subcore_axis_name="s")
    rows_pt = N_PAD // NTILE
    chunk = 1000
    nchunk = E_PT // chunk
    nz = rows_pt // ZROWS

    def body(ones_hbm, s_hbm, z_hbm, out_hbm,
             is0, is1, ones_v, zbuf, acc, ss0, ss1):
        cid = lax.axis_index("c")
        sid = lax.axis_index("s")
        base = sid * rows_pt
        ebase = (cid * NTILE + sid) * E_PT
        pltpu.sync_copy(ones_hbm, ones_v)
        pltpu.sync_copy(z_hbm, zbuf)
        for j in range(nz):
            pltpu.sync_copy(zbuf, acc.at[pl.ds(base + j * ZROWS, ZROWS)])
        plsc.subcore_barrier()
        ibufs = [(is0, ss0), (is1, ss1)]
        pend = [None, None]
        for i in range(nchunk):
            s = i % 2
            is_, ss = ibufs[s]
            if pend[s] is not None:
                pend[s].wait()
                pend[s] = None
            pltpu.sync_copy(s_hbm.at[pl.ds(ebase + i * chunk, chunk)], is_)
            pend[s] = pltpu.async_copy(ones_v, acc.at[is_], ss, add=True)
        for p in pend:
            if p is not None:
                p.wait()
        plsc.subcore_barrier()
        pltpu.sync_copy(acc.at[pl.ds(base, rows_pt)],
                        out_hbm.at[cid, pl.ds(base, rows_pt)])

    return pl.kernel(
        body,
        out_type=jax.ShapeDtypeStruct((NCORE, N_PAD, 16), _f32),
        mesh=mesh,
        scratch_types=[
            pltpu.VMEM((chunk,), jnp.int32),
            pltpu.VMEM((chunk,), jnp.int32),
            pltpu.VMEM((chunk, 16), _f32),
            pltpu.VMEM((ZROWS, 16), _f32),
            pltpu.VMEM_SHARED((N_PAD, 16), _f32),
            pltpu.SemaphoreType.DMA,
            pltpu.SemaphoreType.DMA,
        ],
        compiler_params=pltpu.CompilerParams(use_tc_tiling_on_sc=False),
    )


_BN = 2048


def _mm_all(h, W):
    K, din, dout = W.shape
    nb = N_PAD // _BN

    def body(h_ref, w_ref, y_ref):
        y_ref[0] = jnp.dot(h_ref[...], w_ref[0],
                           preferred_element_type=_f32)

    return pl.pallas_call(
        body,
        grid=(K, nb),
        in_specs=[pl.BlockSpec((_BN, din), lambda k, n: (n, 0)),
                  pl.BlockSpec((1, din, dout), lambda k, n: (k, 0, 0))],
        out_specs=pl.BlockSpec((1, _BN, dout), lambda k, n: (k, n, 0)),
        out_shape=jax.ShapeDtypeStruct((K, N_PAD, dout), _f32),
    )(h, W)


def _scale(dinv, y):
    D = y.shape[1]
    nb = N_PAD // _BN

    def body(d_ref, y_ref, u_ref):
        u_ref[...] = d_ref[...] * y_ref[...]

    return pl.pallas_call(
        body,
        grid=(nb,),
        in_specs=[pl.BlockSpec((_BN, 1), lambda n: (n, 0)),
                  pl.BlockSpec((_BN, D), lambda n: (n, 0))],
        out_specs=pl.BlockSpec((_BN, D), lambda n: (n, 0)),
        out_shape=jax.ShapeDtypeStruct((N_PAD, D), _f32),
    )(dinv, y)


def _step(dinv, y, p, b2):
    D = y.shape[1]
    nb = N_PAD // _BN

    def body(d_ref, y_ref, p_ref, b2_ref, b_ref, u_ref):
        s = p_ref[0] + p_ref[1]
        b = y_ref[...] - 2.0 * d_ref[...] * s - b2_ref[...]
        b_ref[...] = b
        u_ref[...] = d_ref[...] * b

    return pl.pallas_call(
        body,
        grid=(nb,),
        in_specs=[pl.BlockSpec((_BN, 1), lambda n: (n, 0)),
                  pl.BlockSpec((_BN, D), lambda n: (n, 0)),
                  pl.BlockSpec((2, _BN, D), lambda n: (0, n, 0)),
                  pl.BlockSpec((_BN, D), lambda n: (n, 0))],
        out_specs=[pl.BlockSpec((_BN, D), lambda n: (n, 0)),
                   pl.BlockSpec((_BN, D), lambda n: (n, 0))],
        out_shape=[jax.ShapeDtypeStruct((N_PAD, D), _f32),
                   jax.ShapeDtypeStruct((N_PAD, D), _f32)],
    )(dinv, y, p, b2)


def _final(dinv, y, p, b2, gg, be, relu):
    D = y.shape[1]
    nb = N_PAD // _BN

    def body(d_ref, y_ref, p_ref, b2_ref, gg_ref, be_ref, o_ref):
        s = p_ref[0] + p_ref[1]
        o = gg_ref[...] * (y_ref[...] - d_ref[...] * s - b2_ref[...]) \
            + be_ref[...]
        o_ref[...] = jnp.maximum(o, 0.0) if relu else o

    return pl.pallas_call(
        body,
        grid=(nb,),
        in_specs=[pl.BlockSpec((_BN, 1), lambda n: (n, 0)),
                  pl.BlockSpec((_BN, D), lambda n: (n, 0)),
                  pl.BlockSpec((2, _BN, D), lambda n: (0, n, 0)),
                  pl.BlockSpec((_BN, D), lambda n: (n, 0)),
                  pl.BlockSpec((1, D), lambda n: (0, 0)),
                  pl.BlockSpec((1, D), lambda n: (0, 0))],
        out_specs=pl.BlockSpec((_BN, D), lambda n: (n, 0)),
        out_shape=jax.ShapeDtypeStruct((N_PAD, D), _f32),
    )(dinv, y, p, b2, gg, be)


def _pad_to(a, shape):
    pads = [(0, t - s) for s, t in zip(a.shape, shape)]
    return jnp.pad(a, pads)


def _cheb_layer(h, W, bias, g, be, dinv, src, dst, zbuf_hbm, dout_pad, relu):
    K = W.shape[0]
    Wp = _pad_to(W, (K, h.shape[1], dout_pad))
    y = _mm_all(h, Wp)
    if g is not None:
        gg = g / jnp.sqrt(1.0 + 1e-5)
        bev = gg * bias + be
    else:
        gg = jnp.ones_like(bias)
        bev = bias
    gg = _pad_to(gg, (dout_pad,)).reshape(1, dout_pad)
    bev = _pad_to(bev, (dout_pad,)).reshape(1, dout_pad)

    lap = _sc_scatter_gather(dout_pad)
    u = _scale(dinv, y[K - 1])
    bk2 = jnp.zeros((N_PAD, dout_pad), _f32)
    bk1 = y[K - 1]
    for k in range(K - 2, 0, -1):
        p = lap(u, src, dst, zbuf_hbm)
        b, u = _step(dinv, y[k], p, bk2)
        bk2, bk1 = bk1, b
    p = lap(u, src, dst, zbuf_hbm)
    return _final(dinv, y[0], p, bk2, gg, bev, relu)


def kernel(x, edge_index, W1, b1, g1, be1, W2, b2, g2, be2,
           W3, b3, g3, be3, W4, b4):
    src = edge_index[0]
    dst = edge_index[1]
    xp = _pad_to(x, (N_PAD, x.shape[1]))

    z16 = jnp.zeros((ZROWS, 16), _f32)
    ones = jnp.ones((1000, 16), _f32)
    pdeg = _sc_degree()(ones, src, z16)
    deg = pdeg[0, :, 0] + pdeg[1, :, 0]
    dinv = jnp.where(deg > 0, lax.rsqrt(jnp.maximum(deg, 1e-12)), 0.0)
    dinv = dinv[:, None]

    z32 = jnp.zeros((ZROWS, 32), _f32)
    z64 = jnp.zeros((ZROWS, 64), _f32)

    h = _cheb_layer(xp, W1, b1, g1, be1, dinv, src, dst, z64, 64, True)
    h = _cheb_layer(h, W2, b2, g2, be2, dinv, src, dst, z32, 32, True)
    h = _cheb_layer(h, W3, b3, g3, be3, dinv, src, dst, z16, 16, True)
    h = _cheb_layer(h, W4, b4, None, None, dinv, src, dst, z16, 16, False)
    return h[:N_NODES, :10]


# speedup vs baseline: 27.5188x; 1.4333x over previous
import functools

import jax
import jax.numpy as jnp
from jax import lax
from jax.experimental import pallas as pl
from jax.experimental.pallas import tpu as pltpu
from jax.experimental.pallas import tpu_sc as plsc

N_NODES = 10000
N_PAD = 10240
E = 320000
NCORE = 2
NTILE = 16
E_PT = E // (NCORE * NTILE)
ZROWS = 64

_f32 = jnp.float32


@functools.lru_cache(maxsize=None)
def _sc_scatter_gather(D):
    mesh = plsc.VectorSubcoreMesh(core_axis_name="c", subcore_axis_name="s")
    rows_pt = N_PAD // NTILE
    chunk = 400 if D >= 64 else 1000
    nchunk = E_PT // chunk
    nz = rows_pt // ZROWS

    def body(u_hbm, g_hbm, s_hbm, z_hbm, out_hbm,
             ig0, is0, rows0, ig1, is1, rows1, zbuf, acc,
             sg0, ss0, sg1, ss1):
        cid = lax.axis_index("c")
        sid = lax.axis_index("s")
        base = sid * rows_pt
        ebase = (cid * NTILE + sid) * E_PT
        bufs = [(ig0, is0, rows0, sg0, ss0), (ig1, is1, rows1, sg1, ss1)]
        gath = [None, None]
        pend = [None, None]

        def load(i):
            s = i % 2
            ig, is_, rows, sg, _ = bufs[s]
            if pend[s] is not None:
                pend[s].wait()
                pend[s] = None
            eb = ebase + i * chunk
            pltpu.sync_copy(g_hbm.at[pl.ds(eb, chunk)], ig)
            pltpu.sync_copy(s_hbm.at[pl.ds(eb, chunk)], is_)
            gath[s] = pltpu.async_copy(u_hbm.at[ig], rows, sg)

        load(0)
        pltpu.sync_copy(z_hbm, zbuf)
        for j in range(nz):
            pltpu.sync_copy(zbuf, acc.at[pl.ds(base + j * ZROWS, ZROWS)])
        plsc.subcore_barrier()
        for i in range(nchunk):
            s = i % 2
            if i + 1 < nchunk:
                load(i + 1)
            _, is_, rows, _, ss = bufs[s]
            gath[s].wait()
            pend[s] = pltpu.async_copy(rows, acc.at[is_], ss, add=True)
        for p in pend:
            if p is not None:
                p.wait()
        plsc.subcore_barrier()
        pltpu.sync_copy(acc.at[pl.ds(base, rows_pt)],
                        out_hbm.at[cid, pl.ds(base, rows_pt)])

    return pl.kernel(
        body,
        out_type=jax.ShapeDtypeStruct((NCORE, N_PAD, D), _f32),
        mesh=mesh,
        scratch_types=[
            pltpu.VMEM((chunk,), jnp.int32),
            pltpu.VMEM((chunk,), jnp.int32),
            pltpu.VMEM((chunk, D), _f32),
            pltpu.VMEM((chunk,), jnp.int32),
            pltpu.VMEM((chunk,), jnp.int32),
            pltpu.VMEM((chunk, D), _f32),
            pltpu.VMEM((ZROWS, D), _f32),
            pltpu.VMEM_SHARED((N_PAD, D), _f32),
            pltpu.SemaphoreType.DMA,
            pltpu.SemaphoreType.DMA,
            pltpu.SemaphoreType.DMA,
            pltpu.SemaphoreType.DMA,
        ],
        compiler_params=pltpu.CompilerParams(use_tc_tiling_on_sc=False),
    )


def _sc_degree():
    mesh = plsc.VectorSubcoreMesh(core_axis_name="c", subcore_axis_name="s")
    rows_pt = N_PAD // NTILE
    chunk = 1000
    nchunk = E_PT // chunk
    nz = rows_pt // ZROWS

    def body(ones_hbm, s_hbm, z_hbm, out_hbm,
             is0, is1, ones_v, zbuf, acc, ss0, ss1):
        cid = lax.axis_index("c")
        sid = lax.axis_index("s")
        base = sid * rows_pt
        ebase = (cid * NTILE + sid) * E_PT
        pltpu.sync_copy(ones_hbm, ones_v)
        pltpu.sync_copy(z_hbm, zbuf)
        for j in range(nz):
            pltpu.sync_copy(zbuf, acc.at[pl.ds(base + j * ZROWS, ZROWS)])
        plsc.subcore_barrier()
        ibufs = [(is0, ss0), (is1, ss1)]
        pend = [None, None]
        for i in range(nchunk):
            s = i % 2
            is_, ss = ibufs[s]
            if pend[s] is not None:
                pend[s].wait()
                pend[s] = None
            pltpu.sync_copy(s_hbm.at[pl.ds(ebase + i * chunk, chunk)], is_)
            pend[s] = pltpu.async_copy(ones_v, acc.at[is_], ss, add=True)
        for p in pend:
            if p is not None:
                p.wait()
        plsc.subcore_barrier()
        pltpu.sync_copy(acc.at[pl.ds(base, rows_pt)],
                        out_hbm.at[cid, pl.ds(base, rows_pt)])

    return pl.kernel(
        body,
        out_type=jax.ShapeDtypeStruct((NCORE, N_PAD, 16), _f32),
        mesh=mesh,
        scratch_types=[
            pltpu.VMEM((chunk,), jnp.int32),
            pltpu.VMEM((chunk,), jnp.int32),
            pltpu.VMEM((chunk, 16), _f32),
            pltpu.VMEM((ZROWS, 16), _f32),
            pltpu.VMEM_SHARED((N_PAD, 16), _f32),
            pltpu.SemaphoreType.DMA,
            pltpu.SemaphoreType.DMA,
        ],
        compiler_params=pltpu.CompilerParams(use_tc_tiling_on_sc=False),
    )


_BM = 1280


def _mm_all(h, W):
    K, din, dout = W.shape
    nb = h.shape[0] // _BM

    def body(h_ref, w_ref, y_ref):
        y_ref[0] = jnp.dot(h_ref[...], w_ref[0],
                           preferred_element_type=_f32)

    return pl.pallas_call(
        body,
        grid=(K, nb),
        in_specs=[pl.BlockSpec((_BM, din), lambda k, n: (n, 0)),
                  pl.BlockSpec((1, din, dout), lambda k, n: (k, 0, 0))],
        out_specs=pl.BlockSpec((1, _BM, dout), lambda k, n: (k, n, 0)),
        out_shape=jax.ShapeDtypeStruct((K, h.shape[0], dout), _f32),
    )(h, W)


def _scale(dinv, y):
    M = y.shape[0]
    nb = M // _BM

    def body(d_ref, y_ref, u_ref):
        u_ref[...] = d_ref[...] * y_ref[...]

    return pl.pallas_call(
        body,
        grid=(nb,),
        in_specs=[pl.BlockSpec((_BM, 128), lambda n: (n, 0)),
                  pl.BlockSpec((_BM, 128), lambda n: (n, 0))],
        out_specs=pl.BlockSpec((_BM, 128), lambda n: (n, 0)),
        out_shape=jax.ShapeDtypeStruct((M, 128), _f32),
    )(dinv, y)


def _step(dinv, y, p, b2):
    M = y.shape[0]
    nb = M // _BM

    def body(d_ref, y_ref, p_ref, b2_ref, b_ref, u_ref):
        s = p_ref[0] + p_ref[1]
        b = y_ref[...] - 2.0 * d_ref[...] * s - b2_ref[...]
        b_ref[...] = b
        u_ref[...] = d_ref[...] * b

    return pl.pallas_call(
        body,
        grid=(nb,),
        in_specs=[pl.BlockSpec((_BM, 128), lambda n: (n, 0)),
                  pl.BlockSpec((_BM, 128), lambda n: (n, 0)),
                  pl.BlockSpec((2, _BM, 128), lambda n: (0, n, 0)),
                  pl.BlockSpec((_BM, 128), lambda n: (n, 0))],
        out_specs=[pl.BlockSpec((_BM, 128), lambda n: (n, 0)),
                   pl.BlockSpec((_BM, 128), lambda n: (n, 0))],
        out_shape=[jax.ShapeDtypeStruct((M, 128), _f32),
                   jax.ShapeDtypeStruct((M, 128), _f32)],
    )(dinv, y, p, b2)


def _final(dinv, y, p, b2, gg, be, relu):
    M = y.shape[0]
    nb = M // _BM

    def body(d_ref, y_ref, p_ref, b2_ref, gg_ref, be_ref, o_ref):
        s = p_ref[0] + p_ref[1]
        o = gg_ref[...] * (y_ref[...] - d_ref[...] * s - b2_ref[...]) \
            + be_ref[...]
        o_ref[...] = jnp.maximum(o, 0.0) if relu else o

    return pl.pallas_call(
        body,
        grid=(nb,),
        in_specs=[pl.BlockSpec((_BM, 128), lambda n: (n, 0)),
                  pl.BlockSpec((_BM, 128), lambda n: (n, 0)),
                  pl.BlockSpec((2, _BM, 128), lambda n: (0, n, 0)),
                  pl.BlockSpec((_BM, 128), lambda n: (n, 0)),
                  pl.BlockSpec((1, 128), lambda n: (0, 0)),
                  pl.BlockSpec((1, 128), lambda n: (0, 0))],
        out_specs=pl.BlockSpec((_BM, 128), lambda n: (n, 0)),
        out_shape=jax.ShapeDtypeStruct((M, 128), _f32),
    )(dinv, y, p, b2, gg, be)


def _pad_to(a, shape):
    pads = [(0, t - s) for s, t in zip(a.shape, shape)]
    return jnp.pad(a, pads)


def _block_diag(W, P):
    K, din, dout = W.shape
    out = jnp.zeros((K, P * din, P * dout), W.dtype)
    for p in range(P):
        out = out.at[:, p * din:(p + 1) * din,
                     p * dout:(p + 1) * dout].set(W)
    return out


def _cheb_layer(h_pk, W, bias, g, be, dinv_pk, src, dst, zbuf_hbm, D, relu):
    P = 128 // D
    M = N_PAD // P
    K = W.shape[0]
    din = h_pk.shape[1] // P
    Wb = _block_diag(_pad_to(W, (K, din, D)), P)
    y = _mm_all(h_pk, Wb)
    if g is not None:
        gg = g / jnp.sqrt(1.0 + 1e-5)
        bev = gg * bias + be
    else:
        gg = jnp.ones_like(bias)
        bev = bias
    gg = jnp.tile(_pad_to(gg, (D,)), P).reshape(1, 128)
    bev = jnp.tile(_pad_to(bev, (D,)), P).reshape(1, 128)

    lap = _sc_scatter_gather(D)
    u = _scale(dinv_pk, y[K - 1]).reshape(N_PAD, D)
    bk2 = jnp.zeros((M, 128), _f32)
    bk1 = y[K - 1]
    for k in range(K - 2, 0, -1):
        p = lap(u, src, dst, zbuf_hbm).reshape(2, M, 128)
        b, u2 = _step(dinv_pk, y[k], p, bk2)
        u = u2.reshape(N_PAD, D)
        bk2, bk1 = bk1, b
    p = lap(u, src, dst, zbuf_hbm).reshape(2, M, 128)
    return _final(dinv_pk, y[0], p, bk2, gg, bev, relu)


def kernel(x, edge_index, W1, b1, g1, be1, W2, b2, g2, be2,
           W3, b3, g3, be3, W4, b4):
    src = edge_index[0]
    dst = edge_index[1]
    xp = _pad_to(x, (N_PAD, x.shape[1]))

    z16 = jnp.zeros((ZROWS, 16), _f32)
    ones = jnp.ones((1000, 16), _f32)
    pdeg = _sc_degree()(ones, src, z16)
    deg = pdeg[0, :, 0] + pdeg[1, :, 0]
    dinv = jnp.where(deg > 0, lax.rsqrt(jnp.maximum(deg, 1e-12)), 0.0)
    dinv_pk = {P: jnp.repeat(dinv.reshape(N_PAD // P, P), 128 // P, axis=1)
               for P in (2, 4, 8)}

    z32 = jnp.zeros((ZROWS, 32), _f32)
    z64 = jnp.zeros((ZROWS, 64), _f32)

    h = _cheb_layer(xp.reshape(5120, 256), W1, b1, g1, be1, dinv_pk[2],
                    src, dst, z64, 64, True)
    h = _cheb_layer(h.reshape(2560, 256), W2, b2, g2, be2, dinv_pk[4],
                    src, dst, z32, 32, True)
    h = _cheb_layer(h.reshape(1280, 256), W3, b3, g3, be3, dinv_pk[8],
                    src, dst, z16, 16, True)
    h = _cheb_layer(h.reshape(1280, 128), W4, b4, None, None, dinv_pk[8],
                    src, dst, z16, 16, False)
    return h.reshape(N_PAD, 16)[:N_NODES, :10]


# speedup vs baseline: 32.2174x; 1.1707x over previous
import functools

import jax
import jax.numpy as jnp
from jax import lax
from jax.experimental import pallas as pl
from jax.experimental.pallas import tpu as pltpu
from jax.experimental.pallas import tpu_sc as plsc

N_NODES = 10000
N_PAD = 10240
E = 320000
NCORE = 2
NTILE = 16
E_PT = E // (NCORE * NTILE)
ZROWS = 64
DINV_FLOOR = 2.0 ** -60

_f32 = jnp.float32


def _chunk_for(D):
    return 400 if D >= 64 else 1000


@functools.lru_cache(maxsize=None)
def _sc_scatter_gather(D):
    mesh = plsc.VectorSubcoreMesh(core_axis_name="c", subcore_axis_name="s")
    rows_pt = N_PAD // NTILE
    chunk = _chunk_for(D)
    nchunk = E_PT // chunk
    nz = rows_pt // ZROWS

    def body(u_hbm, g_hbm, s_hbm, z_hbm, out_hbm,
             gidx, sidx, rows0, rows1, zbuf, acc, sg0, sg1, ss0, ss1, sz):
        cid = lax.axis_index("c")
        sid = lax.axis_index("s")
        base = sid * rows_pt
        tbase = (cid * NTILE + sid) * nchunk
        pltpu.sync_copy(g_hbm.at[pl.ds(tbase, nchunk)], gidx)
        pltpu.sync_copy(s_hbm.at[pl.ds(tbase, nchunk)], sidx)
        bufs = [(rows0, sg0, ss0), (rows1, sg1, ss1)]
        gath = [None, None]
        pend = [None, None]

        def start(i):
            s = i % 2
            rows, sg, _ = bufs[s]
            if pend[s] is not None:
                pend[s].wait()
                pend[s] = None
            gath[s] = pltpu.async_copy(u_hbm.at[gidx.at[i]], rows, sg)

        start(0)
        pltpu.sync_copy(z_hbm, zbuf)
        zcs = [pltpu.async_copy(zbuf, acc.at[pl.ds(base + j * ZROWS, ZROWS)],
                                sz) for j in range(nz)]
        for zc in zcs:
            zc.wait()
        plsc.subcore_barrier()
        for i in range(nchunk):
            s = i % 2
            if i + 1 < nchunk:
                start(i + 1)
            rows, _, ss = bufs[s]
            gath[s].wait()
            pend[s] = pltpu.async_copy(rows, acc.at[sidx.at[i]], ss, add=True)
        for p in pend:
            if p is not None:
                p.wait()
        plsc.subcore_barrier()
        pltpu.sync_copy(acc.at[pl.ds(base, rows_pt)],
                        out_hbm.at[cid, pl.ds(base, rows_pt)])

    return pl.kernel(
        body,
        out_type=jax.ShapeDtypeStruct((NCORE, N_PAD, D), _f32),
        mesh=mesh,
        scratch_types=[
            pltpu.VMEM((nchunk, chunk), jnp.int32),
            pltpu.VMEM((nchunk, chunk), jnp.int32),
            pltpu.VMEM((chunk, D), _f32),
            pltpu.VMEM((chunk, D), _f32),
            pltpu.VMEM((ZROWS, D), _f32),
            pltpu.VMEM_SHARED((N_PAD, D), _f32),
            pltpu.SemaphoreType.DMA,
            pltpu.SemaphoreType.DMA,
            pltpu.SemaphoreType.DMA,
            pltpu.SemaphoreType.DMA,
            pltpu.SemaphoreType.DMA,
        ],
        compiler_params=pltpu.CompilerParams(use_tc_tiling_on_sc=False),
    )


def _sc_degree():
    mesh = plsc.VectorSubcoreMesh(core_axis_name="c", subcore_axis_name="s")
    rows_pt = N_PAD // NTILE
    chunk = 1000
    nchunk = E_PT // chunk
    nz = rows_pt // ZROWS

    def body(ones_hbm, s_hbm, z_hbm, out_hbm,
             sidx, ones_v, zbuf, acc, ss0, ss1):
        cid = lax.axis_index("c")
        sid = lax.axis_index("s")
        base = sid * rows_pt
        tbase = (cid * NTILE + sid) * nchunk
        pltpu.sync_copy(s_hbm.at[pl.ds(tbase, nchunk)], sidx)
        pltpu.sync_copy(ones_hbm, ones_v)
        pltpu.sync_copy(z_hbm, zbuf)
        for j in range(nz):
            pltpu.sync_copy(zbuf, acc.at[pl.ds(base + j * ZROWS, ZROWS)])
        plsc.subcore_barrier()
        sems = [ss0, ss1]
        pend = [None, None]
        for i in range(nchunk):
            s = i % 2
            if pend[s] is not None:
                pend[s].wait()
                pend[s] = None
            pend[s] = pltpu.async_copy(ones_v, acc.at[sidx.at[i]], sems[s],
                                       add=True)
        for p in pend:
            if p is not None:
                p.wait()
        plsc.subcore_barrier()
        pltpu.sync_copy(acc.at[pl.ds(base, rows_pt)],
                        out_hbm.at[cid, pl.ds(base, rows_pt)])

    return pl.kernel(
        body,
        out_type=jax.ShapeDtypeStruct((NCORE, N_PAD, 16), _f32),
        mesh=mesh,
        scratch_types=[
            pltpu.VMEM((nchunk, chunk), jnp.int32),
            pltpu.VMEM((chunk, 16), _f32),
            pltpu.VMEM((ZROWS, 16), _f32),
            pltpu.VMEM_SHARED((N_PAD, 16), _f32),
            pltpu.SemaphoreType.DMA,
            pltpu.SemaphoreType.DMA,
        ],
        compiler_params=pltpu.CompilerParams(use_tc_tiling_on_sc=False),
    )


_BM = 1280


def _mm_all(h, W, dp):
    K, din, dout = W.shape
    nb = h.shape[0] // _BM

    def body(h_ref, w_ref, d_ref, y_ref):
        z = jnp.dot(h_ref[...], w_ref[0], preferred_element_type=_f32)
        y_ref[0] = jnp.where(pl.program_id(0) == 0, z, d_ref[...] * z)

    return pl.pallas_call(
        body,
        grid=(K, nb),
        in_specs=[pl.BlockSpec((_BM, din), lambda k, n: (n, 0)),
                  pl.BlockSpec((1, din, dout), lambda k, n: (k, 0, 0)),
                  pl.BlockSpec((_BM, 128), lambda k, n: (n, 0))],
        out_specs=pl.BlockSpec((1, _BM, dout), lambda k, n: (k, n, 0)),
        out_shape=jax.ShapeDtypeStruct((K, h.shape[0], dout), _f32),
    )(h, W, dp)


def _step(dp2, y, p, b2):
    M = y.shape[0]
    nb = M // _BM

    def body(d_ref, y_ref, p_ref, b2_ref, b_ref):
        s = p_ref[0] + p_ref[1]
        b_ref[...] = y_ref[...] - 2.0 * d_ref[...] * s - b2_ref[...]

    return pl.pallas_call(
        body,
        grid=(nb,),
        in_specs=[pl.BlockSpec((_BM, 128), lambda n: (n, 0)),
                  pl.BlockSpec((_BM, 128), lambda n: (n, 0)),
                  pl.BlockSpec((2, _BM, 128), lambda n: (0, n, 0)),
                  pl.BlockSpec((_BM, 128), lambda n: (n, 0))],
        out_specs=pl.BlockSpec((_BM, 128), lambda n: (n, 0)),
        out_shape=jax.ShapeDtypeStruct((M, 128), _f32),
    )(dp2, y, p, b2)


def _final(dp, rinv, y, p, b2, gg, be, relu):
    M = y.shape[0]
    nb = M // _BM

    def body(d_ref, r_ref, y_ref, p_ref, b2_ref, gg_ref, be_ref, o_ref):
        s = p_ref[0] + p_ref[1]
        o = gg_ref[...] * (y_ref[...] - d_ref[...] * s
                           - r_ref[...] * b2_ref[...]) + be_ref[...]
        o_ref[...] = jnp.maximum(o, 0.0) if relu else o

    return pl.pallas_call(
        body,
        grid=(nb,),
        in_specs=[pl.BlockSpec((_BM, 128), lambda n: (n, 0)),
                  pl.BlockSpec((_BM, 128), lambda n: (n, 0)),
                  pl.BlockSpec((_BM, 128), lambda n: (n, 0)),
                  pl.BlockSpec((2, _BM, 128), lambda n: (0, n, 0)),
                  pl.BlockSpec((_BM, 128), lambda n: (n, 0)),
                  pl.BlockSpec((1, 128), lambda n: (0, 0)),
                  pl.BlockSpec((1, 128), lambda n: (0, 0))],
        out_specs=pl.BlockSpec((_BM, 128), lambda n: (n, 0)),
        out_shape=jax.ShapeDtypeStruct((M, 128), _f32),
    )(dp, rinv, y, p, b2, gg, be)


def _pad_to(a, shape):
    pads = [(0, t - s) for s, t in zip(a.shape, shape)]
    return jnp.pad(a, pads)


def _block_diag(W, P):
    K, din, dout = W.shape
    eye = jnp.eye(P, dtype=W.dtype)
    return (eye[None, :, None, :, None]
            * W[:, None, :, None, :]).reshape(K, P * din, P * dout)


def _cheb_layer(h_pk, W, bias, g, be, dp_pk, dp2_pk, rinv_pk,
                src2d, dst2d, zbuf_hbm, D, relu):
    P = 128 // D
    M = N_PAD // P
    K = W.shape[0]
    din = h_pk.shape[1] // P
    Wb = _block_diag(_pad_to(W, (K, din, D)), P)
    Y = _mm_all(h_pk, Wb, dp_pk)
    if g is not None:
        gg = g / jnp.sqrt(1.0 + 1e-5)
        bev = gg * bias + be
    else:
        gg = jnp.ones_like(bias)
        bev = bias
    gg = jnp.tile(_pad_to(gg, (D,)), P).reshape(1, 128)
    bev = jnp.tile(_pad_to(bev, (D,)), P).reshape(1, 128)

    lap = _sc_scatter_gather(D)
    Bk1 = Y[K - 1]
    Bk2 = jnp.zeros((M, 128), _f32)
    for k in range(K - 2, 0, -1):
        p = lap(Bk1.reshape(N_PAD, D), src2d, dst2d,
                zbuf_hbm).reshape(2, M, 128)
        B = _step(dp2_pk, Y[k], p, Bk2)
        Bk2, Bk1 = Bk1, B
    p = lap(Bk1.reshape(N_PAD, D), src2d, dst2d, zbuf_hbm).reshape(2, M, 128)
    return _final(dp_pk, rinv_pk, Y[0], p, Bk2, gg, bev, relu)


def kernel(x, edge_index, W1, b1, g1, be1, W2, b2, g2, be2,
           W3, b3, g3, be3, W4, b4):
    src = edge_index[0]
    dst = edge_index[1]
    src400 = src.reshape(E // 400, 400)
    dst400 = dst.reshape(E // 400, 400)
    src1k = src.reshape(E // 1000, 1000)
    dst1k = dst.reshape(E // 1000, 1000)
    xp = _pad_to(x, (N_PAD, x.shape[1]))

    z16 = jnp.zeros((ZROWS, 16), _f32)
    ones = jnp.ones((1000, 16), _f32)
    pdeg = _sc_degree()(ones, src1k, z16)
    deg = pdeg[0, :, 0] + pdeg[1, :, 0]
    dinv = jnp.where(deg > 0, lax.rsqrt(jnp.maximum(deg, 1e-12)), 0.0)
    dp = jnp.maximum(dinv, DINV_FLOOR)
    rinv = 1.0 / dp

    def pk(v, P):
        return jnp.repeat(v.reshape(N_PAD // P, P), 128 // P, axis=1)

    dp_pk = {P: pk(dp, P) for P in (2, 4, 8)}
    dp2_pk = {P: pk(dp * dp, P) for P in (2, 4, 8)}
    rinv_pk = {P: pk(rinv, P) for P in (2, 4, 8)}

    z32 = jnp.zeros((ZROWS, 32), _f32)
    z64 = jnp.zeros((ZROWS, 64), _f32)

    h = _cheb_layer(xp.reshape(5120, 256), W1, b1, g1, be1, dp_pk[2],
                    dp2_pk[2], rinv_pk[2], src400, dst400, z64, 64, True)
    h = _cheb_layer(h.reshape(2560, 256), W2, b2, g2, be2, dp_pk[4],
                    dp2_pk[4], rinv_pk[4], src1k, dst1k, z32, 32, True)
    h = _cheb_layer(h.reshape(1280, 256), W3, b3, g3, be3, dp_pk[8],
                    dp2_pk[8], rinv_pk[8], src1k, dst1k, z16, 16, True)
    h = _cheb_layer(h.reshape(1280, 128), W4, b4, None, None, dp_pk[8],
                    dp2_pk[8], rinv_pk[8], src1k, dst1k, z16, 16, False)
    return h.reshape(N_PAD, 16)[:N_NODES, :10]
